# baseline (speedup 1.0000x reference)
import functools

import jax
import jax.numpy as jnp
from jax import lax
from jax.experimental import pallas as pl
from jax.experimental.pallas import tpu as pltpu
from jax.experimental.pallas import tpu_sc as plsc

B = 1024
K = 16
D = 32
NUM_REL = 32
GRP = 128 // D

_NC = 2
_NS = 16
_NW = _NC * _NS
_BPW = B // _NW


def _sc_gather_body(u4_hbm, v_hbm, v4_hbm, adjc_hbm, usr4_hbm, ent4_hbm,
                    ug_out, vg_out, ac_out, ng_out,
                    u4_v, v_v, v4_v, ug_v, vg_v, ac_v, nbr4_flat,
                    ng_v, sem, sem2):
    wid = lax.axis_index("s") * _NC + lax.axis_index("c")
    base = wid * _BPW
    pltpu.sync_copy(u4_hbm.at[pl.ds(base, _BPW)], u4_v)
    pltpu.sync_copy(v_hbm.at[pl.ds(base, _BPW)], v_v)
    pltpu.sync_copy(v4_hbm.at[pl.ds(base, _BPW)], v4_v)

    c_ue = pltpu.async_copy(usr4_hbm.at[u4_v], ug_v, sem)
    c_vs = pltpu.async_copy(ent4_hbm.at[v4_v], vg_v, sem)
    c_nb = pltpu.async_copy(adjc_hbm.at[v_v], ac_v, sem2)
    c_nb.wait()

    for i in range(_BPW):
        nbr4_flat[pl.ds(i * K, K)] = lax.shift_right_logical(
            ac_v[i, pl.ds(0, K)], 2)

    chunk = 128
    copies = []
    for j in range(0, _BPW * K, chunk):
        copies.append(
            pltpu.async_copy(ent4_hbm.at[nbr4_flat.at[pl.ds(j, chunk)]],
                             ng_v.at[pl.ds(j, chunk)], sem2))
    c_ue.wait()
    c_vs.wait()
    for c in copies:
        c.wait()

    pltpu.sync_copy(ug_v, ug_out.at[pl.ds(base, _BPW)])
    pltpu.sync_copy(vg_v, vg_out.at[pl.ds(base, _BPW)])
    pltpu.sync_copy(ac_v, ac_out.at[pl.ds(base, _BPW)])
    pltpu.sync_copy(ng_v, ng_out.at[pl.ds(base * K, _BPW * K)])


@functools.cache
def _sc_gather_call():
    mesh = plsc.VectorSubcoreMesh(core_axis_name="c", subcore_axis_name="s",
                                  num_cores=_NC, num_subcores=_NS)
    return pl.kernel(
        _sc_gather_body,
        out_type=[
            jax.ShapeDtypeStruct((B, 128), jnp.float32),
            jax.ShapeDtypeStruct((B, 128), jnp.float32),
            jax.ShapeDtypeStruct((B, 128), jnp.int32),
            jax.ShapeDtypeStruct((B * K, 128), jnp.float32),
        ],
        mesh=mesh,
        scratch_types=[
            pltpu.VMEM((_BPW,), jnp.int32),
            pltpu.VMEM((_BPW,), jnp.int32),
            pltpu.VMEM((_BPW,), jnp.int32),
            pltpu.VMEM((_BPW, 128), jnp.float32),
            pltpu.VMEM((_BPW, 128), jnp.float32),
            pltpu.VMEM((_BPW, 128), jnp.int32),
            pltpu.VMEM((_BPW * K,), jnp.int32),
            pltpu.VMEM((_BPW * K, 128), jnp.float32),
            pltpu.SemaphoreType.DMA,
            pltpu.SemaphoreType.DMA,
        ],
    )


def _sel4(grp, off):
    ndim = grp.ndim
    io = lax.broadcasted_iota(jnp.int32, grp.shape, ndim - 2)
    oh = off[..., None, None] == io
    return jnp.sum(jnp.where(oh, grp, 0.0), axis=ndim - 2)


_BT = 128


def _tc_body(u_ref, v_ref, ug_ref, vg_ref, ac_ref, ng_ref,
             rel_emb_ref, w_ref, b_ref, out_ref):
    uo = u_ref[...] & (GRP - 1)
    vo = v_ref[...] & (GRP - 1)
    u_e = _sel4(ug_ref[...].reshape(_BT, GRP, D), uo[:, 0])
    v_self = _sel4(vg_ref[...].reshape(_BT, GRP, D), vo[:, 0])
    logits = lax.dot_general(u_e, rel_emb_ref[...],
                             (((1,), (1,)), ((), ())),
                             preferred_element_type=jnp.float32)
    ac = ac_ref[...]
    nbr = ac[:, :K]
    rel = ac[:, K:2 * K]
    r_iota = lax.broadcasted_iota(jnp.int32, (_BT, K, NUM_REL), 2)
    onehot = rel[:, :, None] == r_iota
    scores = jnp.sum(jnp.where(onehot, logits[:, None, :], 0.0), axis=2)
    scores = jax.nn.softmax(scores, axis=1)
    n_e = _sel4(ng_ref[...].reshape(_BT, K, GRP, D),
                nbr & (GRP - 1))
    e_u = jnp.sum(scores[:, :, None] * n_e, axis=1)
    h = lax.dot_general(e_u + v_self, w_ref[...],
                        (((1,), (1,)), ((), ())),
                        preferred_element_type=jnp.float32)
    v_u = jnp.maximum(h + b_ref[...], 0.0)
    out_ref[...] = jax.nn.sigmoid(
        jnp.sum(u_e * v_u, axis=1, keepdims=True))


_tc_call = pl.pallas_call(
    _tc_body,
    grid=(B // _BT,),
    in_specs=[
        pl.BlockSpec((_BT, 1), lambda i: (i, 0)),
        pl.BlockSpec((_BT, 1), lambda i: (i, 0)),
        pl.BlockSpec((_BT, 128), lambda i: (i, 0)),
        pl.BlockSpec((_BT, 128), lambda i: (i, 0)),
        pl.BlockSpec((_BT, 128), lambda i: (i, 0)),
        pl.BlockSpec((_BT * K, 128), lambda i: (i, 0)),
        pl.BlockSpec((D, NUM_REL), lambda i: (0, 0)),
        pl.BlockSpec((D, D), lambda i: (0, 0)),
        pl.BlockSpec((1, D), lambda i: (0, 0)),
    ],
    out_specs=pl.BlockSpec((_BT, 1), lambda i: (i, 0)),
    out_shape=jax.ShapeDtypeStruct((B, 1), jnp.float32),
)


@jax.jit
def kernel(u, v, adj_ent, adj_rel, usr_emb, ent_emb, rel_emb, W, b):
    usr4 = usr_emb.reshape(-1, 128)
    ent4 = ent_emb.reshape(-1, 128)
    adjc = jnp.pad(jnp.concatenate([adj_ent, adj_rel], axis=1),
                   ((0, 0), (0, 128 - 2 * K)))
    ug, vg, ac, ng = _sc_gather_call()(
        u // GRP, v, v // GRP, adjc, usr4, ent4)
    out = _tc_call(u.reshape(B, 1), v.reshape(B, 1), ug, vg, ac, ng,
                   rel_emb, W, b.reshape(1, D))
    return out.reshape(B)
